# Initial kernel scaffold; baseline (speedup 1.0000x reference)
import jax
import jax.numpy as jnp
from jax.experimental import pallas as pl


def kernel(x, edge_index, W1, b1, W2, b2):
    raise NotImplementedError("write your pallas kernel here")


# speedup vs baseline: 10.4059x; 10.4059x over previous
import functools

import jax
import jax.numpy as jnp
from jax import lax
from jax.experimental import pallas as pl
from jax.experimental.pallas import tpu as pltpu
from jax.experimental.pallas import tpu_sc as plsc

N = 10000
NP = 10240
D = 128
E = 320000
NC, NS, L = 2, 16, 16
B = 128
NB = 80
EPAD = NC * NS * NB * B
RPT = NP // NS
DEGW = 16
RB = 1280
GRID = NP // RB


@functools.cache
def _sc_kernels():
    mesh = plsc.VectorSubcoreMesh(core_axis_name="c", subcore_axis_name="s",
                                  num_cores=NC, num_subcores=NS)

    @functools.partial(
        pl.kernel,
        out_type=jax.ShapeDtypeStruct((NC, NP, DEGW), jnp.float32),
        mesh=mesh,
        scratch_types=[
            pltpu.VMEM((NB, B), jnp.int32),
            pltpu.VMEM((B, DEGW), jnp.float32),
            pltpu.VMEM((B, DEGW), jnp.float32),
            pltpu.VMEM_SHARED((NP, DEGW), jnp.float32),
        ],
    )
    def _sc_degree(dst_hbm, out_hbm, dst_v, ones_v, zero_v, acc):
        c = lax.axis_index("c")
        s = lax.axis_index("s")
        pltpu.sync_copy(dst_hbm.at[c, s], dst_v)

        def fill(i, carry):
            ones_v[i, :] = jnp.full((DEGW,), 1.0, jnp.float32)
            zero_v[i, :] = jnp.zeros((DEGW,), jnp.float32)
            return carry
        lax.fori_loop(0, B, fill, 0)

        base = s * RPT
        for k in range(RPT // B):
            pltpu.sync_copy(zero_v, acc.at[pl.ds(base + k * B, B)])
        plsc.subcore_barrier()

        def body(j, carry):
            pltpu.sync_copy(ones_v, acc.at[dst_v.at[j]], add=True)
            return carry
        lax.fori_loop(0, NB, body, 0)
        plsc.subcore_barrier()

        for k in range(RPT // B):
            pltpu.sync_copy(acc.at[pl.ds(base + k * B, B)],
                            out_hbm.at[c, pl.ds(base + k * B, B)])

    @functools.partial(
        pl.kernel,
        out_type=jax.ShapeDtypeStruct((NC, NP, D), jnp.float32),
        mesh=mesh,
        scratch_types=[
            pltpu.VMEM((NB // 2, B), jnp.int32),
            pltpu.VMEM((NB // 2, B), jnp.int32),
            pltpu.VMEM((B, D), jnp.float32),
            pltpu.VMEM((B, D), jnp.float32),
            pltpu.VMEM_SHARED((NP, D), jnp.float32),
            pltpu.SemaphoreType.DMA,
            pltpu.SemaphoreType.DMA,
        ],
    )
    def _sc_aggregate(hs_hbm, src_hbm, dst_hbm, out_hbm,
                      src_v, dst_v, rb0, rb1, acc, sem0, sem1):
        c = lax.axis_index("c")
        s = lax.axis_index("s")
        NH = NB // 2

        def zrow(i, carry):
            for jj in range(D // L):
                rb0[i, pl.ds(jj * L, L)] = jnp.zeros((L,), jnp.float32)
            return carry
        lax.fori_loop(0, B, zrow, 0)

        base = s * RPT
        for k in range(RPT // B):
            pltpu.sync_copy(rb0, acc.at[pl.ds(base + k * B, B)])
        plsc.subcore_barrier()

        for h in range(2):
            pltpu.sync_copy(src_hbm.at[c, s, pl.ds(h * NH, NH)], src_v)
            pltpu.sync_copy(dst_hbm.at[c, s, pl.ds(h * NH, NH)], dst_v)
            pltpu.async_copy(hs_hbm.at[src_v.at[0]], rb0, sem0)
            pltpu.async_copy(hs_hbm.at[src_v.at[1]], rb1, sem1)

            def body(i, carry):
                j = 2 * i
                pltpu.make_async_copy(hs_hbm.at[src_v.at[j]], rb0, sem0).wait()
                pltpu.sync_copy(rb0, acc.at[dst_v.at[j]], add=True)
                pltpu.async_copy(hs_hbm.at[src_v.at[j + 2]], rb0, sem0)
                pltpu.make_async_copy(hs_hbm.at[src_v.at[j + 1]], rb1, sem1).wait()
                pltpu.sync_copy(rb1, acc.at[dst_v.at[j + 1]], add=True)
                pltpu.async_copy(hs_hbm.at[src_v.at[j + 3]], rb1, sem1)
                return carry
            lax.fori_loop(0, NH // 2 - 1, body, 0)

            j = NH - 2
            pltpu.make_async_copy(hs_hbm.at[src_v.at[j]], rb0, sem0).wait()
            pltpu.sync_copy(rb0, acc.at[dst_v.at[j]], add=True)
            pltpu.make_async_copy(hs_hbm.at[src_v.at[j + 1]], rb1, sem1).wait()
            pltpu.sync_copy(rb1, acc.at[dst_v.at[j + 1]], add=True)

        plsc.subcore_barrier()
        for k in range(RPT // B):
            pltpu.sync_copy(acc.at[pl.ds(base + k * B, B)],
                            out_hbm.at[c, pl.ds(base + k * B, B)])

    return _sc_degree, _sc_aggregate


def _dis_block(degp_ref):
    deg = degp_ref[0] + degp_ref[1] + 1.0
    dis = lax.rsqrt(deg)
    return jnp.broadcast_to(dis[:, :1], (RB, D))


def _tc_layer1(xp, W1, degp):
    def body(x_ref, w_ref, degp_ref, t_ref, hs_ref):
        t = jnp.dot(x_ref[...], w_ref[...], preferred_element_type=jnp.float32)
        disb = _dis_block(degp_ref)
        t_ref[...] = t
        hs_ref[...] = t * disb

    return pl.pallas_call(
        body,
        grid=(GRID,),
        in_specs=[
            pl.BlockSpec((RB, D), lambda i: (i, 0)),
            pl.BlockSpec((D, D), lambda i: (0, 0)),
            pl.BlockSpec((NC, RB, DEGW), lambda i: (0, i, 0)),
        ],
        out_specs=[
            pl.BlockSpec((RB, D), lambda i: (i, 0)),
            pl.BlockSpec((RB, D), lambda i: (i, 0)),
        ],
        out_shape=[
            jax.ShapeDtypeStruct((NP, D), jnp.float32),
            jax.ShapeDtypeStruct((NP, D), jnp.float32),
        ],
    )(xp, W1, degp)


def _tc_layer2(aggp, degp, t1, W2, b1):
    def body(agg_ref, degp_ref, t1_ref, w_ref, b_ref, t2_ref, hs_ref):
        i = pl.program_id(0)
        disb = _dis_block(degp_ref)
        q = agg_ref[0] + agg_ref[1]
        z = disb * q + disb * disb * t1_ref[...] + b_ref[...]
        h = jnp.maximum(z, 0.0)
        t2 = jnp.dot(h, w_ref[...], preferred_element_type=jnp.float32)
        rows = i * RB + lax.broadcasted_iota(jnp.int32, (RB, D), 0)
        t2_ref[...] = t2
        hs_ref[...] = jnp.where(rows < N, t2 * disb, 0.0)

    return pl.pallas_call(
        body,
        grid=(GRID,),
        in_specs=[
            pl.BlockSpec((NC, RB, D), lambda i: (0, i, 0)),
            pl.BlockSpec((NC, RB, DEGW), lambda i: (0, i, 0)),
            pl.BlockSpec((RB, D), lambda i: (i, 0)),
            pl.BlockSpec((D, D), lambda i: (0, 0)),
            pl.BlockSpec((1, D), lambda i: (0, 0)),
        ],
        out_specs=[
            pl.BlockSpec((RB, D), lambda i: (i, 0)),
            pl.BlockSpec((RB, D), lambda i: (i, 0)),
        ],
        out_shape=[
            jax.ShapeDtypeStruct((NP, D), jnp.float32),
            jax.ShapeDtypeStruct((NP, D), jnp.float32),
        ],
    )(aggp, degp, t1, W2, b1)


def _tc_layer3(aggp, degp, t2, b2):
    def body(agg_ref, degp_ref, t2_ref, b_ref, out_ref):
        disb = _dis_block(degp_ref)
        q = agg_ref[0] + agg_ref[1]
        out_ref[...] = disb * q + disb * disb * t2_ref[...] + b_ref[...]

    return pl.pallas_call(
        body,
        grid=(GRID,),
        in_specs=[
            pl.BlockSpec((NC, RB, D), lambda i: (0, i, 0)),
            pl.BlockSpec((NC, RB, DEGW), lambda i: (0, i, 0)),
            pl.BlockSpec((RB, D), lambda i: (i, 0)),
            pl.BlockSpec((1, D), lambda i: (0, 0)),
        ],
        out_specs=pl.BlockSpec((RB, D), lambda i: (i, 0)),
        out_shape=jax.ShapeDtypeStruct((NP, D), jnp.float32),
    )(aggp, degp, t2, b2)


def kernel(x, edge_index, W1, b1, W2, b2):
    sc_degree, sc_aggregate = _sc_kernels()
    pad = EPAD - E
    srcp = jnp.concatenate(
        [edge_index[0], jnp.full((pad,), N, jnp.int32)]).reshape(NC, NS, NB, B)
    dstp = jnp.concatenate(
        [edge_index[1], jnp.full((pad,), N, jnp.int32)]).reshape(NC, NS, NB, B)
    xp = jnp.pad(x, ((0, NP - N), (0, 0)))

    degp = sc_degree(dstp)
    t1, hs1 = _tc_layer1(xp, W1, degp)
    aggp1 = sc_aggregate(hs1, srcp, dstp)
    t2, hs2 = _tc_layer2(aggp1, degp, t1, W2, b1.reshape(1, D))
    aggp2 = sc_aggregate(hs2, srcp, dstp)
    outp = _tc_layer3(aggp2, degp, t2, b2.reshape(1, D))
    return outp[:N]


# speedup vs baseline: 25.7917x; 2.4786x over previous
import functools

import jax
import jax.numpy as jnp
from jax import lax
from jax.experimental import pallas as pl
from jax.experimental.pallas import tpu as pltpu
from jax.experimental.pallas import tpu_sc as plsc

N = 10000
NP = 10240
D = 128
E = 320000
NC, NS, L = 2, 16, 16
B = 128
NB = 80
EPAD = NC * NS * NB * B
RPT = NP // NS
DEGW = 16
RB = 1280
GRID = NP // RB


@functools.cache
def _sc_kernels():
    mesh = plsc.VectorSubcoreMesh(core_axis_name="c", subcore_axis_name="s",
                                  num_cores=NC, num_subcores=NS)

    @functools.partial(
        pl.kernel,
        out_type=jax.ShapeDtypeStruct((NC, NP, DEGW), jnp.float32),
        mesh=mesh,
        scratch_types=[
            pltpu.VMEM((NB, B), jnp.int32),
            pltpu.VMEM((B, DEGW), jnp.float32),
            pltpu.VMEM((B, DEGW), jnp.float32),
            pltpu.VMEM_SHARED((NP, DEGW), jnp.float32),
        ],
    )
    def _sc_degree(dst_hbm, out_hbm, dst_v, ones_v, zero_v, acc):
        c = lax.axis_index("c")
        s = lax.axis_index("s")
        pltpu.sync_copy(dst_hbm.at[c, s], dst_v)

        def fill(i, carry):
            ones_v[i, :] = jnp.full((DEGW,), 1.0, jnp.float32)
            zero_v[i, :] = jnp.zeros((DEGW,), jnp.float32)
            return carry
        lax.fori_loop(0, B, fill, 0)

        base = s * RPT
        for k in range(RPT // B):
            pltpu.sync_copy(zero_v, acc.at[pl.ds(base + k * B, B)])
        plsc.subcore_barrier()

        def body(j, carry):
            pltpu.sync_copy(ones_v, acc.at[dst_v.at[j]], add=True)
            return carry
        lax.fori_loop(0, NB, body, 0)
        plsc.subcore_barrier()

        for k in range(RPT // B):
            pltpu.sync_copy(acc.at[pl.ds(base + k * B, B)],
                            out_hbm.at[c, pl.ds(base + k * B, B)])

    @functools.partial(
        pl.kernel,
        out_type=jax.ShapeDtypeStruct((NC, NP, D), jnp.float32),
        mesh=mesh,
        scratch_types=[
            pltpu.VMEM((NB // 2, B), jnp.int32),
            pltpu.VMEM((NB // 2, B), jnp.int32),
            pltpu.VMEM((B, D), jnp.float32),
            pltpu.VMEM((B, D), jnp.float32),
            pltpu.VMEM_SHARED((NP, D), jnp.float32),
            pltpu.SemaphoreType.DMA,
            pltpu.SemaphoreType.DMA,
            pltpu.SemaphoreType.DMA,
            pltpu.SemaphoreType.DMA,
        ],
    )
    def _sc_aggregate(hs_hbm, src_hbm, dst_hbm, out_hbm,
                      src_v, dst_v, rb0, rb1, acc, g0, g1, s0, s1):
        c = lax.axis_index("c")
        s = lax.axis_index("s")
        NH = NB // 2

        def zrow(i, carry):
            for jj in range(D // L):
                rb0[i, pl.ds(jj * L, L)] = jnp.zeros((L,), jnp.float32)
            return carry
        lax.fori_loop(0, B, zrow, 0)

        base = s * RPT
        for k in range(RPT // B):
            pltpu.sync_copy(rb0, acc.at[pl.ds(base + k * B, B)])
        plsc.subcore_barrier()

        for h in range(2):
            pltpu.sync_copy(src_hbm.at[c, s, pl.ds(h * NH, NH)], src_v)
            pltpu.sync_copy(dst_hbm.at[c, s, pl.ds(h * NH, NH)], dst_v)
            pltpu.async_copy(hs_hbm.at[src_v.at[0]], rb0, g0)
            pltpu.async_copy(hs_hbm.at[src_v.at[1]], rb1, g1)

            def body(i, carry):
                j = 2 * i
                pltpu.make_async_copy(hs_hbm.at[src_v.at[j]], rb0, g0).wait()
                pltpu.async_copy(rb0, acc.at[dst_v.at[j]], s0, add=True)
                pltpu.make_async_copy(
                    hs_hbm.at[src_v.at[j + 1]], rb1, g1).wait()
                pltpu.async_copy(rb1, acc.at[dst_v.at[j + 1]], s1, add=True)
                pltpu.make_async_copy(
                    rb0, acc.at[dst_v.at[j]], s0).wait()
                pltpu.async_copy(hs_hbm.at[src_v.at[j + 2]], rb0, g0)
                pltpu.make_async_copy(
                    rb1, acc.at[dst_v.at[j + 1]], s1).wait()
                pltpu.async_copy(hs_hbm.at[src_v.at[j + 3]], rb1, g1)
                return carry
            lax.fori_loop(0, NH // 2 - 1, body, 0)

            j = NH - 2
            pltpu.make_async_copy(hs_hbm.at[src_v.at[j]], rb0, g0).wait()
            pltpu.async_copy(rb0, acc.at[dst_v.at[j]], s0, add=True)
            pltpu.make_async_copy(hs_hbm.at[src_v.at[j + 1]], rb1, g1).wait()
            pltpu.async_copy(rb1, acc.at[dst_v.at[j + 1]], s1, add=True)
            pltpu.make_async_copy(rb0, acc.at[dst_v.at[j]], s0).wait()
            pltpu.make_async_copy(rb1, acc.at[dst_v.at[j + 1]], s1).wait()

        plsc.subcore_barrier()
        for k in range(RPT // B):
            pltpu.sync_copy(acc.at[pl.ds(base + k * B, B)],
                            out_hbm.at[c, pl.ds(base + k * B, B)])

    return _sc_degree, _sc_aggregate


def _dis_block(degp_ref):
    deg = degp_ref[0] + degp_ref[1] + 1.0
    dis = lax.rsqrt(deg)
    return jnp.broadcast_to(dis[:, :1], (RB, D))


def _tc_layer1(xp, W1, degp):
    def body(x_ref, w_ref, degp_ref, t_ref, hs_ref):
        t = jnp.dot(x_ref[...], w_ref[...], preferred_element_type=jnp.float32)
        disb = _dis_block(degp_ref)
        t_ref[...] = t
        hs_ref[...] = t * disb

    return pl.pallas_call(
        body,
        grid=(GRID,),
        in_specs=[
            pl.BlockSpec((RB, D), lambda i: (i, 0)),
            pl.BlockSpec((D, D), lambda i: (0, 0)),
            pl.BlockSpec((NC, RB, DEGW), lambda i: (0, i, 0)),
        ],
        out_specs=[
            pl.BlockSpec((RB, D), lambda i: (i, 0)),
            pl.BlockSpec((RB, D), lambda i: (i, 0)),
        ],
        out_shape=[
            jax.ShapeDtypeStruct((NP, D), jnp.float32),
            jax.ShapeDtypeStruct((NP, D), jnp.float32),
        ],
    )(xp, W1, degp)


def _tc_layer2(aggp, degp, t1, W2, b1):
    def body(agg_ref, degp_ref, t1_ref, w_ref, b_ref, t2_ref, hs_ref):
        i = pl.program_id(0)
        disb = _dis_block(degp_ref)
        q = agg_ref[0] + agg_ref[1]
        z = disb * q + disb * disb * t1_ref[...] + b_ref[...]
        h = jnp.maximum(z, 0.0)
        t2 = jnp.dot(h, w_ref[...], preferred_element_type=jnp.float32)
        rows = i * RB + lax.broadcasted_iota(jnp.int32, (RB, D), 0)
        t2_ref[...] = t2
        hs_ref[...] = jnp.where(rows < N, t2 * disb, 0.0)

    return pl.pallas_call(
        body,
        grid=(GRID,),
        in_specs=[
            pl.BlockSpec((NC, RB, D), lambda i: (0, i, 0)),
            pl.BlockSpec((NC, RB, DEGW), lambda i: (0, i, 0)),
            pl.BlockSpec((RB, D), lambda i: (i, 0)),
            pl.BlockSpec((D, D), lambda i: (0, 0)),
            pl.BlockSpec((1, D), lambda i: (0, 0)),
        ],
        out_specs=[
            pl.BlockSpec((RB, D), lambda i: (i, 0)),
            pl.BlockSpec((RB, D), lambda i: (i, 0)),
        ],
        out_shape=[
            jax.ShapeDtypeStruct((NP, D), jnp.float32),
            jax.ShapeDtypeStruct((NP, D), jnp.float32),
        ],
    )(aggp, degp, t1, W2, b1)


def _tc_layer3(aggp, degp, t2, b2):
    def body(agg_ref, degp_ref, t2_ref, b_ref, out_ref):
        disb = _dis_block(degp_ref)
        q = agg_ref[0] + agg_ref[1]
        out_ref[...] = disb * q + disb * disb * t2_ref[...] + b_ref[...]

    return pl.pallas_call(
        body,
        grid=(GRID,),
        in_specs=[
            pl.BlockSpec((NC, RB, D), lambda i: (0, i, 0)),
            pl.BlockSpec((NC, RB, DEGW), lambda i: (0, i, 0)),
            pl.BlockSpec((RB, D), lambda i: (i, 0)),
            pl.BlockSpec((1, D), lambda i: (0, 0)),
        ],
        out_specs=pl.BlockSpec((RB, D), lambda i: (i, 0)),
        out_shape=jax.ShapeDtypeStruct((NP, D), jnp.float32),
    )(aggp, degp, t2, b2)


def kernel(x, edge_index, W1, b1, W2, b2):
    sc_degree, sc_aggregate = _sc_kernels()
    pad = EPAD - E
    pad_idx = N + jnp.arange(pad, dtype=jnp.int32) % (NP - N)
    srcp = jnp.concatenate(
        [edge_index[0], pad_idx]).reshape(NC, NS, NB, B)
    dstp = jnp.concatenate(
        [edge_index[1], pad_idx]).reshape(NC, NS, NB, B)
    xp = jnp.pad(x, ((0, NP - N), (0, 0)))

    degp = sc_degree(dstp)
    t1, hs1 = _tc_layer1(xp, W1, degp)
    aggp1 = sc_aggregate(hs1, srcp, dstp)
    t2, hs2 = _tc_layer2(aggp1, degp, t1, W2, b1.reshape(1, D))
    aggp2 = sc_aggregate(hs2, srcp, dstp)
    outp = _tc_layer3(aggp2, degp, t2, b2.reshape(1, D))
    return outp[:N]


# speedup vs baseline: 29.1978x; 1.1321x over previous
import functools

import jax
import jax.numpy as jnp
from jax import lax
from jax.experimental import pallas as pl
from jax.experimental.pallas import tpu as pltpu
from jax.experimental.pallas import tpu_sc as plsc

N = 10000
NP = 10240
D = 128
E = 320000
NC, NS, L = 2, 16, 16
B = 80
NB = 128
NQ = 4
NBQ = NB // NQ
NR = 4
EPAD = NC * NS * NB * B
RPT = NP // NS
DEGW = 16
RB = 1280
GRID = NP // RB


@functools.cache
def _sc_kernels():
    mesh = plsc.VectorSubcoreMesh(core_axis_name="c", subcore_axis_name="s",
                                  num_cores=NC, num_subcores=NS)

    @functools.partial(
        pl.kernel,
        out_type=jax.ShapeDtypeStruct((NC, NP, DEGW), jnp.float32),
        mesh=mesh,
        scratch_types=[
            pltpu.VMEM((NB, B), jnp.int32),
            pltpu.VMEM((B, DEGW), jnp.float32),
            pltpu.VMEM((B, DEGW), jnp.float32),
            pltpu.VMEM_SHARED((NP, DEGW), jnp.float32),
        ],
    )
    def _sc_degree(dst_hbm, out_hbm, dst_v, ones_v, zero_v, acc):
        c = lax.axis_index("c")
        s = lax.axis_index("s")
        pltpu.sync_copy(dst_hbm.at[c, s], dst_v)

        def fill(i, carry):
            ones_v[i, :] = jnp.full((DEGW,), 1.0, jnp.float32)
            zero_v[i, :] = jnp.zeros((DEGW,), jnp.float32)
            return carry
        lax.fori_loop(0, B, fill, 0)

        base = s * RPT
        for k in range(RPT // B):
            pltpu.sync_copy(zero_v, acc.at[pl.ds(base + k * B, B)])
        plsc.subcore_barrier()

        def body(j, carry):
            pltpu.sync_copy(ones_v, acc.at[dst_v.at[j]], add=True)
            return carry
        lax.fori_loop(0, NB, body, 0)
        plsc.subcore_barrier()

        for k in range(RPT // B):
            pltpu.sync_copy(acc.at[pl.ds(base + k * B, B)],
                            out_hbm.at[c, pl.ds(base + k * B, B)])

    @functools.partial(
        pl.kernel,
        out_type=jax.ShapeDtypeStruct((NC, NP, D), jnp.float32),
        mesh=mesh,
        scratch_types=(
            [pltpu.VMEM((NBQ, B), jnp.int32),
             pltpu.VMEM((NBQ, B), jnp.int32)]
            + [pltpu.VMEM((B, D), jnp.float32) for _ in range(NR)]
            + [pltpu.VMEM_SHARED((NP, D), jnp.float32)]
            + [pltpu.SemaphoreType.DMA for _ in range(2 * NR)]
        ),
    )
    def _sc_aggregate(hs_hbm, src_hbm, dst_hbm, out_hbm,
                      src_v, dst_v, *rest):
        rb = rest[:NR]
        acc = rest[NR]
        gsem = rest[NR + 1:NR + 1 + NR]
        ssem = rest[NR + 1 + NR:]
        c = lax.axis_index("c")
        s = lax.axis_index("s")

        def zrow(i, carry):
            for jj in range(D // L):
                rb[0][i, pl.ds(jj * L, L)] = jnp.zeros((L,), jnp.float32)
            return carry
        lax.fori_loop(0, B, zrow, 0)

        base = s * RPT
        for k in range(RPT // B):
            pltpu.sync_copy(rb[0], acc.at[pl.ds(base + k * B, B)])
        plsc.subcore_barrier()

        def gather(j, b):
            pltpu.async_copy(hs_hbm.at[src_v.at[j]], rb[b], gsem[b])

        def wait_gather(j, b):
            pltpu.make_async_copy(hs_hbm.at[src_v.at[j]], rb[b],
                                  gsem[b]).wait()

        def scatter(j, b):
            pltpu.async_copy(rb[b], acc.at[dst_v.at[j]], ssem[b], add=True)

        def wait_scatter(j, b):
            pltpu.make_async_copy(rb[b], acc.at[dst_v.at[j]],
                                  ssem[b]).wait()

        for q in range(NQ):
            pltpu.sync_copy(src_hbm.at[c, s, pl.ds(q * NBQ, NBQ)], src_v)
            pltpu.sync_copy(dst_hbm.at[c, s, pl.ds(q * NBQ, NBQ)], dst_v)
            for jj in range(NR):
                gather(jj, jj)

            def body(g, carry):
                j0 = g * NR
                for jj in range(NR):
                    wait_gather(j0 + jj, jj)
                    scatter(j0 + jj, jj)
                for jj in range(NR):
                    wait_scatter(j0 + jj, jj)
                    gather(j0 + NR + jj, jj)
                return carry
            lax.fori_loop(0, NBQ // NR - 1, body, 0)

            j0 = NBQ - NR
            for jj in range(NR):
                wait_gather(j0 + jj, jj)
                scatter(j0 + jj, jj)
            for jj in range(NR):
                wait_scatter(j0 + jj, jj)

        plsc.subcore_barrier()
        for k in range(RPT // B):
            pltpu.sync_copy(acc.at[pl.ds(base + k * B, B)],
                            out_hbm.at[c, pl.ds(base + k * B, B)])

    return _sc_degree, _sc_aggregate


def _dis_block(degp_ref):
    deg = degp_ref[0] + degp_ref[1] + 1.0
    dis = lax.rsqrt(deg)
    return jnp.broadcast_to(dis[:, :1], (RB, D))


def _tc_layer1(xp, W1, degp):
    def body(x_ref, w_ref, degp_ref, t_ref, hs_ref):
        t = jnp.dot(x_ref[...], w_ref[...], preferred_element_type=jnp.float32)
        disb = _dis_block(degp_ref)
        t_ref[...] = t
        hs_ref[...] = t * disb

    return pl.pallas_call(
        body,
        grid=(GRID,),
        in_specs=[
            pl.BlockSpec((RB, D), lambda i: (i, 0)),
            pl.BlockSpec((D, D), lambda i: (0, 0)),
            pl.BlockSpec((NC, RB, DEGW), lambda i: (0, i, 0)),
        ],
        out_specs=[
            pl.BlockSpec((RB, D), lambda i: (i, 0)),
            pl.BlockSpec((RB, D), lambda i: (i, 0)),
        ],
        out_shape=[
            jax.ShapeDtypeStruct((NP, D), jnp.float32),
            jax.ShapeDtypeStruct((NP, D), jnp.float32),
        ],
    )(xp, W1, degp)


def _tc_layer2(aggp, degp, t1, W2, b1):
    def body(agg_ref, degp_ref, t1_ref, w_ref, b_ref, t2_ref, hs_ref):
        i = pl.program_id(0)
        disb = _dis_block(degp_ref)
        q = agg_ref[0] + agg_ref[1]
        z = disb * q + disb * disb * t1_ref[...] + b_ref[...]
        h = jnp.maximum(z, 0.0)
        t2 = jnp.dot(h, w_ref[...], preferred_element_type=jnp.float32)
        rows = i * RB + lax.broadcasted_iota(jnp.int32, (RB, D), 0)
        t2_ref[...] = t2
        hs_ref[...] = jnp.where(rows < N, t2 * disb, 0.0)

    return pl.pallas_call(
        body,
        grid=(GRID,),
        in_specs=[
            pl.BlockSpec((NC, RB, D), lambda i: (0, i, 0)),
            pl.BlockSpec((NC, RB, DEGW), lambda i: (0, i, 0)),
            pl.BlockSpec((RB, D), lambda i: (i, 0)),
            pl.BlockSpec((D, D), lambda i: (0, 0)),
            pl.BlockSpec((1, D), lambda i: (0, 0)),
        ],
        out_specs=[
            pl.BlockSpec((RB, D), lambda i: (i, 0)),
            pl.BlockSpec((RB, D), lambda i: (i, 0)),
        ],
        out_shape=[
            jax.ShapeDtypeStruct((NP, D), jnp.float32),
            jax.ShapeDtypeStruct((NP, D), jnp.float32),
        ],
    )(aggp, degp, t1, W2, b1)


def _tc_layer3(aggp, degp, t2, b2):
    def body(agg_ref, degp_ref, t2_ref, b_ref, out_ref):
        disb = _dis_block(degp_ref)
        q = agg_ref[0] + agg_ref[1]
        out_ref[...] = disb * q + disb * disb * t2_ref[...] + b_ref[...]

    return pl.pallas_call(
        body,
        grid=(GRID,),
        in_specs=[
            pl.BlockSpec((NC, RB, D), lambda i: (0, i, 0)),
            pl.BlockSpec((NC, RB, DEGW), lambda i: (0, i, 0)),
            pl.BlockSpec((RB, D), lambda i: (i, 0)),
            pl.BlockSpec((1, D), lambda i: (0, 0)),
        ],
        out_specs=pl.BlockSpec((RB, D), lambda i: (i, 0)),
        out_shape=jax.ShapeDtypeStruct((NP, D), jnp.float32),
    )(aggp, degp, t2, b2)


def kernel(x, edge_index, W1, b1, W2, b2):
    sc_degree, sc_aggregate = _sc_kernels()
    pad = EPAD - E
    pad_idx = N + jnp.arange(pad, dtype=jnp.int32) % (NP - N)
    srcp = jnp.concatenate(
        [edge_index[0], pad_idx]).reshape(NC, NS, NB, B)
    dstp = jnp.concatenate(
        [edge_index[1], pad_idx]).reshape(NC, NS, NB, B)
    xp = jnp.pad(x, ((0, NP - N), (0, 0)))

    degp = sc_degree(dstp)
    t1, hs1 = _tc_layer1(xp, W1, degp)
    aggp1 = sc_aggregate(hs1, srcp, dstp)
    t2, hs2 = _tc_layer2(aggp1, degp, t1, W2, b1.reshape(1, D))
    aggp2 = sc_aggregate(hs2, srcp, dstp)
    outp = _tc_layer3(aggp2, degp, t2, b2.reshape(1, D))
    return outp[:N]
